# baseline (speedup 1.0000x reference)
import jax
import jax.numpy as jnp
from jax import lax
from jax.experimental import pallas as pl
from jax.experimental.pallas import tpu as pltpu
from jax.experimental.pallas import tpu_sc as plsc

B = 4096
L = 200
LP = L + 1
D = 32
V = 1000000
NC = 2
NS = 16
NW = NC * NS
NBG = B // 128
NLG = L // 8
HV = 4


def _body(x4, tabin, out4, idx_v, rows0, rows1, slab0, slab1, zbuf,
          sg0, sg1, so0, so1):
    w = lax.axis_index("s") * NC + lax.axis_index("c")
    tab = tabin
    rows = (rows0, rows1)
    slab = (slab0, slab1)
    sg = (sg0, sg1)
    so = (so0, so1)

    pltpu.sync_copy(x4.at[:, w], idx_v)

    zero16 = jnp.zeros((16,), jnp.float32)

    def zb(t, c):
        ds = t >> 3
        j = t & 7
        zbuf[ds, pl.ds(j * 16, 16)] = zero16
        return c

    lax.fori_loop(0, 64, zb, 0)
    for dg in range(4):
        pltpu.sync_copy(zbuf, out4.at[L, dg, pl.ds(w * 8, 8)])

    rbase = [lax.iota(jnp.int32, 16) + (blk * 16) for blk in range(8)]

    def fire(lg, h, s):
        for lv in range(HV):
            ls = h * HV + lv
            pltpu.async_copy(tab.at[idx_v.at[lg, ls]],
                             rows[s].at[pl.ds(lv * 128, 128)], sg[s])

    def drain_g(s):
        for lv in range(HV):
            pltpu.make_async_copy(tab.at[idx_v.at[0, 0]],
                                  rows[s].at[pl.ds(lv * 128, 128)],
                                  sg[s]).wait()

    def start_wb(lg, h, s):
        l0 = lg * 8 + h * HV
        for dg in range(4):
            pltpu.async_copy(slab[s].at[:, pl.ds(dg * 8, 8)],
                             out4.at[pl.ds(l0, HV), dg, pl.ds(w * 8, 8)],
                             so[s])

    def drain_wb(s):
        for dg in range(4):
            pltpu.make_async_copy(slab[s].at[:, pl.ds(dg * 8, 8)],
                                  out4.at[pl.ds(0, HV), dg, pl.ds(w * 8, 8)],
                                  so[s]).wait()

    def transpose(s):
        r2 = rows[s]
        sl = slab[s]

        def lvd(t, c):
            lv = t >> 5
            d = t & 31
            rb = jnp.full((16,), lv * 128, jnp.int32)
            dv = jnp.full((16,), d, jnp.int32)
            for blk in range(8):
                vals = plsc.load_gather(r2, [rbase[blk] + rb, dv])
                sl[lv, d, pl.ds(blk * 16, 16)] = vals
            return c

        lax.fori_loop(0, HV * D, lvd, 0)

    fire(0, 0, 0)
    drain_g(0)
    fire(0, 1, 1)
    transpose(0)
    start_wb(0, 0, 0)
    drain_g(1)
    fire(1, 0, 0)
    transpose(1)
    start_wb(0, 1, 1)

    def rnd(rr, c):
        lg = rr + 1
        drain_g(0)
        fire(lg, 1, 1)
        drain_wb(0)
        transpose(0)
        start_wb(lg, 0, 0)
        drain_g(1)
        fire(lg + 1, 0, 0)
        drain_wb(1)
        transpose(1)
        start_wb(lg, 1, 1)
        return c

    lax.fori_loop(0, NLG - 2, rnd, 0)

    lg = NLG - 1
    drain_g(0)
    fire(lg, 1, 1)
    drain_wb(0)
    transpose(0)
    start_wb(lg, 0, 0)
    drain_g(1)
    drain_wb(1)
    transpose(1)
    start_wb(lg, 1, 1)
    drain_wb(0)
    drain_wb(1)


@jax.jit
def _lookup_pad(x4, tabin):
    mesh = plsc.VectorSubcoreMesh(core_axis_name="c", subcore_axis_name="s")
    f = pl.kernel(
        _body,
        out_type=jax.ShapeDtypeStruct((LP, 4, NBG * 8, 128), jnp.float32),
        mesh=mesh,
        scratch_types=[
            pltpu.VMEM((NLG, 8, 128), jnp.int32),
            pltpu.VMEM((HV * 128, D), jnp.float32),
            pltpu.VMEM((HV * 128, D), jnp.float32),
            pltpu.VMEM((HV, D, 128), jnp.float32),
            pltpu.VMEM((HV, D, 128), jnp.float32),
            pltpu.VMEM((8, 128), jnp.float32),
            pltpu.SemaphoreType.DMA,
            pltpu.SemaphoreType.DMA,
            pltpu.SemaphoreType.DMA,
            pltpu.SemaphoreType.DMA,
        ],
        compiler_params=pltpu.CompilerParams(
            use_tc_tiling_on_sc=False, needs_layout_passes=False
        ),
    )
    return f(x4, tabin)


def kernel(x, y, pad_id, embed_x, embed_y):
    x4 = x.astype(jnp.int32).reshape(32, 128, NLG, 8).transpose(2, 0, 3, 1)
    out4 = _lookup_pad(x4, embed_x)
    out5 = out4.reshape(LP, 4, NBG, 8, 128)
    return out5.transpose(2, 4, 0, 1, 3).reshape(B, LP, D)


# speedup vs baseline: 1.3481x; 1.3481x over previous
import jax
import jax.numpy as jnp
from jax import lax
from jax.experimental import pallas as pl
from jax.experimental.pallas import tpu as pltpu
from jax.experimental.pallas import tpu_sc as plsc

B = 4096
L = 200
LP = L + 1
D = 32
V = 1000000
NC = 2
NS = 16
NW = NC * NS
NBG = B // 128
NLG = L // 8
HV = 4


def _body(x4, tabin, out4, idx_v, rows0, rows1, slab0, slab1, zbuf,
          sg0, sg1, so0, so1):
    w = lax.axis_index("s") * NC + lax.axis_index("c")
    tab = tabin
    rows = (rows0, rows1)
    slab = (slab0, slab1)
    sg = (sg0, sg1)
    so = (so0, so1)

    pltpu.sync_copy(x4.at[:, w], idx_v)

    zero16 = jnp.zeros((16,), jnp.float32)

    def zb(t, c):
        ds = t >> 3
        j = t & 7
        zbuf[ds, pl.ds(j * 16, 16)] = zero16
        return c

    lax.fori_loop(0, 64, zb, 0)
    for dg in range(4):
        pltpu.sync_copy(zbuf, out4.at[L, dg, pl.ds(w * 8, 8)])

    rbase = [lax.iota(jnp.int32, 16) + (blk * 16) for blk in range(8)]

    def fire(lg, h, s):
        for lv in range(HV):
            ls = h * HV + lv
            pltpu.async_copy(tab.at[idx_v.at[lg, ls]],
                             rows[s].at[pl.ds(lv * 128, 128)], sg[s])

    def drain_g(s):
        for lv in range(HV):
            pltpu.make_async_copy(tab.at[idx_v.at[0, 0]],
                                  rows[s].at[pl.ds(lv * 128, 128)],
                                  sg[s]).wait()

    def start_wb(lg, h, s):
        l0 = lg * 8 + h * HV
        for dg in range(4):
            pltpu.async_copy(slab[s].at[:, pl.ds(dg * 8, 8)],
                             out4.at[pl.ds(l0, HV), dg, pl.ds(w * 8, 8)],
                             so[s])

    def drain_wb(s):
        for dg in range(4):
            pltpu.make_async_copy(slab[s].at[:, pl.ds(dg * 8, 8)],
                                  out4.at[pl.ds(0, HV), dg, pl.ds(w * 8, 8)],
                                  so[s]).wait()

    def transpose(s):
        r2 = rows[s]
        sl = slab[s]

        @plsc.parallel_loop(0, HV * D, 1, unroll=4)
        def lvd(t):
            lv = t >> 5
            d = t & 31
            rb = jnp.full((16,), lv * 128, jnp.int32)
            dv = jnp.full((16,), d, jnp.int32)
            for blk in range(8):
                vals = plsc.load_gather(r2, [rbase[blk] + rb, dv])
                sl[lv, d, pl.ds(blk * 16, 16)] = vals

    fire(0, 0, 0)
    drain_g(0)
    fire(0, 1, 1)
    transpose(0)
    start_wb(0, 0, 0)
    drain_g(1)
    fire(1, 0, 0)
    transpose(1)
    start_wb(0, 1, 1)

    def rnd(rr, c):
        lg = rr + 1
        drain_g(0)
        fire(lg, 1, 1)
        drain_wb(0)
        transpose(0)
        start_wb(lg, 0, 0)
        drain_g(1)
        fire(lg + 1, 0, 0)
        drain_wb(1)
        transpose(1)
        start_wb(lg, 1, 1)
        return c

    lax.fori_loop(0, NLG - 2, rnd, 0)

    lg = NLG - 1
    drain_g(0)
    fire(lg, 1, 1)
    drain_wb(0)
    transpose(0)
    start_wb(lg, 0, 0)
    drain_g(1)
    drain_wb(1)
    transpose(1)
    start_wb(lg, 1, 1)
    drain_wb(0)
    drain_wb(1)


@jax.jit
def _lookup_pad(x4, tabin):
    mesh = plsc.VectorSubcoreMesh(core_axis_name="c", subcore_axis_name="s")
    f = pl.kernel(
        _body,
        out_type=jax.ShapeDtypeStruct((LP, 4, NBG * 8, 128), jnp.float32),
        mesh=mesh,
        scratch_types=[
            pltpu.VMEM((NLG, 8, 128), jnp.int32),
            pltpu.VMEM((HV * 128, D), jnp.float32),
            pltpu.VMEM((HV * 128, D), jnp.float32),
            pltpu.VMEM((HV, D, 128), jnp.float32),
            pltpu.VMEM((HV, D, 128), jnp.float32),
            pltpu.VMEM((8, 128), jnp.float32),
            pltpu.SemaphoreType.DMA,
            pltpu.SemaphoreType.DMA,
            pltpu.SemaphoreType.DMA,
            pltpu.SemaphoreType.DMA,
        ],
        compiler_params=pltpu.CompilerParams(
            use_tc_tiling_on_sc=False,
            needs_layout_passes=False,
            disable_bounds_checks=True,
        ),
    )
    return f(x4, tabin)


def kernel(x, y, pad_id, embed_x, embed_y):
    x4 = x.astype(jnp.int32).reshape(32, 128, NLG, 8).transpose(2, 0, 3, 1)
    out4 = _lookup_pad(x4, embed_x)
    out5 = out4.reshape(LP, 4, NBG, 8, 128)
    return out5.transpose(2, 4, 0, 1, 3).reshape(B, LP, D)
